# baseline (speedup 1.0000x reference)
import functools

import jax
import jax.numpy as jnp
from jax.experimental import pallas as pl
from jax.experimental.pallas import tpu as pltpu

_N = 10000
_BM = 400


def _proj_kernel(x_ref, gc1_w_ref, lin1_w_ref, lin1_b_ref, gc2_w_ref,
                 sab_ref, l1_ref):
    x = x_ref[...]
    sA = jnp.dot(x, gc1_w_ref[...], preferred_element_type=jnp.float32)
    l1 = jnp.dot(x, lin1_w_ref[...], preferred_element_type=jnp.float32)
    l1 = l1 + lin1_b_ref[...]
    sB = jnp.dot(l1, gc2_w_ref[...], preferred_element_type=jnp.float32)
    sab_ref[...] = jnp.concatenate([sA, sB], axis=1)
    l1_ref[...] = l1


def _pass1_kernel(adj_ref, sab_ref, l1_ref, gc1_b_ref, gc2_b_ref,
                  gc2_w_ref, lin2_w_ref, lin2_b_ref,
                  sc_ref, hbb_ref, u_ref):
    hab = jnp.dot(adj_ref[...], sab_ref[...],
                  preferred_element_type=jnp.float32)
    hA = hab[:, :16]
    hB = hab[:, 16:]
    r1 = jnp.maximum(hA + gc1_b_ref[...], 0.0)
    sc_ref[...] = jnp.dot(r1, gc2_w_ref[...],
                          preferred_element_type=jnp.float32)
    hbb_ref[...] = hB + gc2_b_ref[...]
    u_ref[...] = (jnp.dot(r1 + l1_ref[...], lin2_w_ref[...],
                          preferred_element_type=jnp.float32)
                  + lin2_b_ref[...])


def _pass2_kernel(adj_ref, sc_ref, hbb_ref, u_ref, out_ref):
    hc = jnp.dot(adj_ref[...], sc_ref[...],
                 preferred_element_type=jnp.float32)
    out_ref[...] = jnp.maximum(hc + hbb_ref[...], 0.0) + u_ref[...]


@jax.jit
def kernel(x, adj, gc1_w, gc1_b, gc2_w, gc2_b,
           lin1_w, lin1_b, lin2_w, lin2_b):
    n, nfeat = x.shape
    nhid = gc1_w.shape[1]
    ncls = gc2_w.shape[1]

    gc1_b2 = gc1_b.reshape(1, nhid)
    gc2_b2 = gc2_b.reshape(1, ncls)
    lin1_b2 = lin1_b.reshape(1, nhid)
    lin2_b2 = lin2_b.reshape(1, ncls)

    sab, l1 = pl.pallas_call(
        _proj_kernel,
        out_shape=(
            jax.ShapeDtypeStruct((n, nhid + ncls), jnp.float32),
            jax.ShapeDtypeStruct((n, nhid), jnp.float32),
        ),
    )(x, gc1_w, lin1_w, lin1_b2, gc2_w)

    grid = (n // _BM,)
    row_spec = pl.BlockSpec((_BM, n), lambda i: (i, 0))
    full = lambda r, c: pl.BlockSpec((r, c), lambda i: (0, 0))
    blk = lambda c: pl.BlockSpec((_BM, c), lambda i: (i, 0))

    sc, hbb, u = pl.pallas_call(
        _pass1_kernel,
        grid=grid,
        in_specs=[
            row_spec,
            full(n, nhid + ncls),
            blk(nhid),
            full(1, nhid),
            full(1, ncls),
            full(nhid, ncls),
            full(nhid, ncls),
            full(1, ncls),
        ],
        out_specs=(blk(ncls), blk(ncls), blk(ncls)),
        out_shape=(
            jax.ShapeDtypeStruct((n, ncls), jnp.float32),
            jax.ShapeDtypeStruct((n, ncls), jnp.float32),
            jax.ShapeDtypeStruct((n, ncls), jnp.float32),
        ),
        compiler_params=pltpu.CompilerParams(
            dimension_semantics=("arbitrary",),
        ),
    )(adj, sab, l1, gc1_b2, gc2_b2, gc2_w, lin2_w, lin2_b2)

    out = pl.pallas_call(
        _pass2_kernel,
        grid=grid,
        in_specs=[row_spec, full(n, ncls), blk(ncls), blk(ncls)],
        out_specs=blk(ncls),
        out_shape=jax.ShapeDtypeStruct((n, ncls), jnp.float32),
        compiler_params=pltpu.CompilerParams(
            dimension_semantics=("arbitrary",),
        ),
    )(adj, sc, hbb, u)
    return out


# speedup vs baseline: 1.0656x; 1.0656x over previous
import jax
import jax.numpy as jnp
from jax.experimental import pallas as pl
from jax.experimental.pallas import tpu as pltpu

_BM = 200


def _gcn_kernel(x_ref, adj_ref, gc1_w_ref, gc1_b_ref, gc2_w_ref, gc2_b_ref,
                lin1_w_ref, lin1_b_ref, lin2_w_ref, lin2_b_ref,
                out_ref, sab_s, sc_s, misc_s):
    j = pl.program_id(0)
    i = pl.program_id(1)
    bm = adj_ref.shape[0]
    nhid = gc1_w_ref.shape[1]
    rows = pl.ds(i * bm, bm)

    @pl.when(jnp.logical_and(j == 0, i == 0))
    def _proj():
        xx = x_ref[...]
        sA = jnp.dot(xx, gc1_w_ref[...], preferred_element_type=jnp.float32)
        l1 = jnp.dot(xx, lin1_w_ref[...], preferred_element_type=jnp.float32)
        l1 = l1 + lin1_b_ref[...]
        sB = jnp.dot(l1, gc2_w_ref[...], preferred_element_type=jnp.float32)
        sab_s[...] = jnp.concatenate([sA, sB], axis=1)
        misc_s[:, 0:8] = (jnp.dot(l1, lin2_w_ref[...],
                                  preferred_element_type=jnp.float32)
                          + lin2_b_ref[...])

    @pl.when(j == 0)
    def _pass1():
        hab = jnp.dot(adj_ref[...], sab_s[...],
                      preferred_element_type=jnp.float32)
        r1 = jnp.maximum(hab[:, :nhid] + gc1_b_ref[...], 0.0)
        sc_s[rows, :] = jnp.dot(r1, gc2_w_ref[...],
                                preferred_element_type=jnp.float32)
        misc_s[rows, 8:16] = hab[:, nhid:] + gc2_b_ref[...]
        misc_s[rows, 16:24] = (jnp.dot(r1, lin2_w_ref[...],
                                       preferred_element_type=jnp.float32)
                               + misc_s[rows, 0:8])

    @pl.when(j == 1)
    def _pass2():
        hc = jnp.dot(adj_ref[...], sc_s[...],
                     preferred_element_type=jnp.float32)
        out_ref[...] = (jnp.maximum(hc + misc_s[rows, 8:16], 0.0)
                        + misc_s[rows, 16:24])


@jax.jit
def kernel(x, adj, gc1_w, gc1_b, gc2_w, gc2_b,
           lin1_w, lin1_b, lin2_w, lin2_b):
    n, nfeat = x.shape
    nhid = gc1_w.shape[1]
    ncls = gc2_w.shape[1]

    full = lambda r, c: pl.BlockSpec((r, c), lambda j, i: (0, 0))

    out = pl.pallas_call(
        _gcn_kernel,
        grid=(2, n // _BM),
        in_specs=[
            full(n, nfeat),
            pl.BlockSpec((_BM, n), lambda j, i: (i, 0)),
            full(nfeat, nhid),
            full(1, nhid),
            full(nhid, ncls),
            full(1, ncls),
            full(nfeat, nhid),
            full(1, nhid),
            full(nhid, ncls),
            full(1, ncls),
        ],
        out_specs=pl.BlockSpec((_BM, ncls), lambda j, i: (i, 0)),
        out_shape=jax.ShapeDtypeStruct((n, ncls), jnp.float32),
        scratch_shapes=[
            pltpu.VMEM((n, nhid + ncls), jnp.float32),
            pltpu.VMEM((n, ncls), jnp.float32),
            pltpu.VMEM((n, 3 * ncls), jnp.float32),
        ],
        compiler_params=pltpu.CompilerParams(
            dimension_semantics=("arbitrary", "arbitrary"),
        ),
    )(x, adj, gc1_w, gc1_b.reshape(1, nhid), gc2_w, gc2_b.reshape(1, ncls),
      lin1_w, lin1_b.reshape(1, nhid), lin2_w, lin2_b.reshape(1, ncls))
    return out
